# baseline (speedup 1.0000x reference)
import jax
import jax.numpy as jnp
from jax import lax
from jax.experimental import pallas as pl
from jax.experimental.pallas import tpu as pltpu
from jax.experimental.pallas import tpu_sc as plsc

_NC = 2
_NS = 16
_NW = _NC * _NS
_LANES = 16

_E = 320000
_D = 128
_VPR = _D // _LANES
_PER_W = _E // _NW
_CHUNK = 48
_NFULL = _PER_W // _CHUNK
_TAIL = _PER_W - _NFULL * _CHUNK
_IVPC = _CHUNK // _LANES

_NROW = 10000
_ZSTAGE = 624

_SHUF_DNUMS = lax.GatherDimensionNumbers(
    offset_dims=(), collapsed_slice_dims=(0,), start_index_map=(0,))


def _shuffle(v, perm):
    return lax.gather(v, perm[:, None], _SHUF_DNUMS, slice_sizes=(1,),
                      mode=lax.GatherScatterMode.PROMISE_IN_BOUNDS)


def _softmax_rows(n, src_rows, dst_rows, out_rows):
    @plsc.parallel_loop(0, n, unroll=4)
    def edge_body(e):
        ex = [jnp.exp(src_rows[e, pl.ds(j * _LANES, _LANES)]
                      * dst_rows[e, pl.ds(j * _LANES, _LANES)])
              for j in range(_VPR)]
        t = [ex[2 * j] + ex[2 * j + 1] for j in range(4)]
        u = [t[0] + t[1], t[2] + t[3]]
        s = u[0] + u[1]
        for sh in (8, 4, 2, 1):
            perm = lax.iota(jnp.int32, 16) ^ sh
            s = s + _shuffle(s, perm)
        r = 1.0 / s
        for j in range(_VPR):
            out_rows[e, pl.ds(j * _LANES, _LANES)] = ex[j] * r


def _sc_body(z_hbm, pidx_hbm, out_hbm,
             z_sp, pidx_v, si0, si1, di0, di1,
             sr0, sr1, dr0, dr1, or0, or1,
             ss0, ss1, sd0, sd1, sw0, sw1):
    sid = lax.axis_index("s")
    wid = sid * _NC + lax.axis_index("c")
    w_base = wid * _PER_W

    pltpu.sync_copy(z_hbm.at[pl.ds(sid * _ZSTAGE, _ZSTAGE)],
                    z_sp.at[pl.ds(sid * _ZSTAGE, _ZSTAGE)])

    @pl.when(sid == 0)
    def _():
        pltpu.sync_copy(z_hbm.at[pl.ds(_NS * _ZSTAGE, _NROW - _NS * _ZSTAGE)],
                        z_sp.at[pl.ds(_NS * _ZSTAGE, _NROW - _NS * _ZSTAGE)])

    pltpu.sync_copy(pidx_hbm.at[pl.ds(w_base, _PER_W)], pidx_v)
    plsc.subcore_barrier()

    src_idx = (si0, si1)
    dst_idx = (di0, di1)
    src_rows = (sr0, sr1)
    dst_rows = (dr0, dr1)
    out_rows = (or0, or1)
    sem_s = (ss0, ss1)
    sem_d = (sd0, sd1)
    sem_w = (sw0, sw1)

    def unpack_idx(i, b, n_vregs):
        for k in range(n_vregs):
            w = pidx_v[pl.ds(i * _CHUNK + k * _LANES, _LANES)]
            src_idx[b][pl.ds(k * _LANES, _LANES)] = w & 0xFFFF
            dst_idx[b][pl.ds(k * _LANES, _LANES)] = (
                lax.shift_right_logical(w, 16))

    def gather(i, b):
        unpack_idx(i, b, _IVPC)
        pltpu.make_async_copy(
            z_sp.at[src_idx[b]], src_rows[b], sem_s[b]).start()
        pltpu.make_async_copy(
            z_sp.at[dst_idx[b]], dst_rows[b], sem_d[b]).start()

    def wait_gather(b):
        pltpu.make_async_copy(
            z_sp.at[src_idx[b]], src_rows[b], sem_s[b]).wait()
        pltpu.make_async_copy(
            z_sp.at[dst_idx[b]], dst_rows[b], sem_d[b]).wait()

    def writeback(i, b):
        pltpu.make_async_copy(
            out_rows[b], out_hbm.at[pl.ds(w_base + i * _CHUNK, _CHUNK)],
            sem_w[b]).start()

    def wait_writeback(i, b):
        pltpu.make_async_copy(
            out_rows[b], out_hbm.at[pl.ds(w_base + i * _CHUNK, _CHUNK)],
            sem_w[b]).wait()

    gather(0, 0)

    def pair_body(p, carry):
        for b in (0, 1):
            i = 2 * p + b

            @pl.when(i + 1 < _NFULL)
            def _():
                gather(i + 1, 1 - b)

            wait_gather(b)

            @pl.when(i >= 2)
            def _():
                wait_writeback(i - 2, b)

            _softmax_rows(_CHUNK, src_rows[b], dst_rows[b], out_rows[b])
            writeback(i, b)
        return carry

    lax.fori_loop(0, _NFULL // 2, pair_body, 0)

    t_base = w_base + _NFULL * _CHUNK
    unpack_idx(_NFULL, 0, _TAIL // _LANES)
    pltpu.async_copy(
        z_sp.at[si0.at[pl.ds(0, _TAIL)]], sr0.at[pl.ds(0, _TAIL)], ss0).wait()
    pltpu.async_copy(
        z_sp.at[di0.at[pl.ds(0, _TAIL)]], dr0.at[pl.ds(0, _TAIL)], sd0).wait()
    wait_writeback(_NFULL - 2, 0)
    _softmax_rows(_TAIL, sr0, dr0, or0)
    pltpu.sync_copy(or0.at[pl.ds(0, _TAIL)], out_hbm.at[pl.ds(t_base, _TAIL)])
    wait_writeback(_NFULL - 1, 1)


def _decode(z, packed_idx):
    mesh = plsc.VectorSubcoreMesh(core_axis_name="c", subcore_axis_name="s",
                                  num_cores=_NC, num_subcores=_NS)
    return pl.kernel(
        _sc_body,
        out_type=jax.ShapeDtypeStruct((_E, _D), jnp.float32),
        mesh=mesh,
        scratch_types=[
            pltpu.VMEM_SHARED((_NROW, _D), jnp.float32),
            pltpu.VMEM((_PER_W,), jnp.int32),
            pltpu.VMEM((_CHUNK,), jnp.int32),
            pltpu.VMEM((_CHUNK,), jnp.int32),
            pltpu.VMEM((_CHUNK,), jnp.int32),
            pltpu.VMEM((_CHUNK,), jnp.int32),
            pltpu.VMEM((_CHUNK, _D), jnp.float32),
            pltpu.VMEM((_CHUNK, _D), jnp.float32),
            pltpu.VMEM((_CHUNK, _D), jnp.float32),
            pltpu.VMEM((_CHUNK, _D), jnp.float32),
            pltpu.VMEM((_CHUNK, _D), jnp.float32),
            pltpu.VMEM((_CHUNK, _D), jnp.float32),
            pltpu.SemaphoreType.DMA,
            pltpu.SemaphoreType.DMA,
            pltpu.SemaphoreType.DMA,
            pltpu.SemaphoreType.DMA,
            pltpu.SemaphoreType.DMA,
            pltpu.SemaphoreType.DMA,
        ],
    )(z, packed_idx)


def kernel(z, edge_index):
    ei = edge_index.astype(jnp.int32)
    packed = ei[0] | (ei[1] << 16)
    return _decode(z, packed)


# speedup vs baseline: 1.1305x; 1.1305x over previous
import jax
import jax.numpy as jnp
from jax import lax
from jax.experimental import pallas as pl
from jax.experimental.pallas import tpu as pltpu
from jax.experimental.pallas import tpu_sc as plsc

_NC = 2
_NS = 16
_NW = _NC * _NS
_LANES = 16

_E = 320000
_D = 128
_VPR = _D // _LANES
_PER_W = _E // _NW
_CHUNK = 48
_NFULL = _PER_W // _CHUNK
_TAIL = _PER_W - _NFULL * _CHUNK
_IVPC = _CHUNK // _LANES

_NROW = 10000
_ZSTAGE = 624

_SHUF_DNUMS = lax.GatherDimensionNumbers(
    offset_dims=(), collapsed_slice_dims=(0,), start_index_map=(0,))


def _shuffle(v, perm):
    return lax.gather(v, perm[:, None], _SHUF_DNUMS, slice_sizes=(1,),
                      mode=lax.GatherScatterMode.PROMISE_IN_BOUNDS)


def _softmax_rows(n, src_rows, dst_rows, out_rows):
    @plsc.parallel_loop(0, n, unroll=4)
    def edge_body(e):
        ex = [jnp.exp(jnp.minimum(
            src_rows[e, pl.ds(j * _LANES, _LANES)]
            * dst_rows[e, pl.ds(j * _LANES, _LANES)], 80.0))
            for j in range(_VPR)]
        t = [ex[2 * j] + ex[2 * j + 1] for j in range(4)]
        u = [t[0] + t[1], t[2] + t[3]]
        s = u[0] + u[1]
        for sh in (8, 4, 2, 1):
            perm = lax.iota(jnp.int32, 16) ^ sh
            s = s + _shuffle(s, perm)
        r = 1.0 / s
        for j in range(_VPR):
            out_rows[e, pl.ds(j * _LANES, _LANES)] = ex[j] * r


def _sc_body(z_hbm, pidx_hbm, out_hbm,
             z_sp, pidx_v, si0, si1, di0, di1,
             sr0, sr1, dr0, dr1, or0, or1,
             ss0, ss1, sd0, sd1, sw0, sw1):
    sid = lax.axis_index("s")
    wid = sid * _NC + lax.axis_index("c")
    w_base = wid * _PER_W

    pltpu.sync_copy(z_hbm.at[pl.ds(sid * _ZSTAGE, _ZSTAGE)],
                    z_sp.at[pl.ds(sid * _ZSTAGE, _ZSTAGE)])

    @pl.when(sid == 0)
    def _():
        pltpu.sync_copy(z_hbm.at[pl.ds(_NS * _ZSTAGE, _NROW - _NS * _ZSTAGE)],
                        z_sp.at[pl.ds(_NS * _ZSTAGE, _NROW - _NS * _ZSTAGE)])

    pltpu.sync_copy(pidx_hbm.at[pl.ds(w_base, _PER_W)], pidx_v)
    plsc.subcore_barrier()

    src_idx = (si0, si1)
    dst_idx = (di0, di1)
    src_rows = (sr0, sr1)
    dst_rows = (dr0, dr1)
    out_rows = (or0, or1)
    sem_s = (ss0, ss1)
    sem_d = (sd0, sd1)
    sem_w = (sw0, sw1)

    def unpack_idx(i, b, n_vregs):
        for k in range(n_vregs):
            w = pidx_v[pl.ds(i * _CHUNK + k * _LANES, _LANES)]
            src_idx[b][pl.ds(k * _LANES, _LANES)] = w & 0xFFFF
            dst_idx[b][pl.ds(k * _LANES, _LANES)] = (
                lax.shift_right_logical(w, 16))

    def gather(i, b):
        unpack_idx(i, b, _IVPC)
        pltpu.make_async_copy(
            z_sp.at[src_idx[b]], src_rows[b], sem_s[b]).start()
        pltpu.make_async_copy(
            z_sp.at[dst_idx[b]], dst_rows[b], sem_d[b]).start()

    def wait_gather(b):
        pltpu.make_async_copy(
            z_sp.at[src_idx[b]], src_rows[b], sem_s[b]).wait()
        pltpu.make_async_copy(
            z_sp.at[dst_idx[b]], dst_rows[b], sem_d[b]).wait()

    def writeback(i, b):
        pltpu.make_async_copy(
            out_rows[b], out_hbm.at[pl.ds(w_base + i * _CHUNK, _CHUNK)],
            sem_w[b]).start()

    def wait_writeback(i, b):
        pltpu.make_async_copy(
            out_rows[b], out_hbm.at[pl.ds(w_base + i * _CHUNK, _CHUNK)],
            sem_w[b]).wait()

    gather(0, 0)

    def pair_body(p, carry):
        for b in (0, 1):
            i = 2 * p + b

            @pl.when(i + 1 < _NFULL)
            def _():
                gather(i + 1, 1 - b)

            wait_gather(b)

            @pl.when(i >= 2)
            def _():
                wait_writeback(i - 2, b)

            _softmax_rows(_CHUNK, src_rows[b], dst_rows[b], out_rows[b])
            writeback(i, b)
        return carry

    lax.fori_loop(0, _NFULL // 2, pair_body, 0)

    t_base = w_base + _NFULL * _CHUNK
    unpack_idx(_NFULL, 0, _TAIL // _LANES)
    pltpu.async_copy(
        z_sp.at[si0.at[pl.ds(0, _TAIL)]], sr0.at[pl.ds(0, _TAIL)], ss0).wait()
    pltpu.async_copy(
        z_sp.at[di0.at[pl.ds(0, _TAIL)]], dr0.at[pl.ds(0, _TAIL)], sd0).wait()
    wait_writeback(_NFULL - 2, 0)
    _softmax_rows(_TAIL, sr0, dr0, or0)
    pltpu.sync_copy(or0.at[pl.ds(0, _TAIL)], out_hbm.at[pl.ds(t_base, _TAIL)])
    wait_writeback(_NFULL - 1, 1)


def _decode(z, packed_idx):
    mesh = plsc.VectorSubcoreMesh(core_axis_name="c", subcore_axis_name="s",
                                  num_cores=_NC, num_subcores=_NS)
    return pl.kernel(
        _sc_body,
        out_type=jax.ShapeDtypeStruct((_E, _D), jnp.float32),
        mesh=mesh,
        scratch_types=[
            pltpu.VMEM_SHARED((_NROW, _D), jnp.float32),
            pltpu.VMEM((_PER_W,), jnp.int32),
            pltpu.VMEM((_CHUNK,), jnp.int32),
            pltpu.VMEM((_CHUNK,), jnp.int32),
            pltpu.VMEM((_CHUNK,), jnp.int32),
            pltpu.VMEM((_CHUNK,), jnp.int32),
            pltpu.VMEM((_CHUNK, _D), jnp.float32),
            pltpu.VMEM((_CHUNK, _D), jnp.float32),
            pltpu.VMEM((_CHUNK, _D), jnp.float32),
            pltpu.VMEM((_CHUNK, _D), jnp.float32),
            pltpu.VMEM((_CHUNK, _D), jnp.float32),
            pltpu.VMEM((_CHUNK, _D), jnp.float32),
            pltpu.SemaphoreType.DMA,
            pltpu.SemaphoreType.DMA,
            pltpu.SemaphoreType.DMA,
            pltpu.SemaphoreType.DMA,
            pltpu.SemaphoreType.DMA,
            pltpu.SemaphoreType.DMA,
        ],
    )(z, packed_idx)


def kernel(z, edge_index):
    ei = edge_index.astype(jnp.int32)
    packed = ei[0] | (ei[1] << 16)
    return _decode(z, packed)


# speedup vs baseline: 1.1642x; 1.0298x over previous
import jax
import jax.numpy as jnp
from jax import lax
from jax.experimental import pallas as pl
from jax.experimental.pallas import tpu as pltpu
from jax.experimental.pallas import tpu_sc as plsc

_NC = 2
_NS = 16
_NW = _NC * _NS
_LANES = 16

_E = 320000
_D = 128
_VPR = _D // _LANES
_PER_W = _E // _NW
_CHUNK = 48
_NFULL = _PER_W // _CHUNK
_TAIL = _PER_W - _NFULL * _CHUNK
_IVPC = _CHUNK // _LANES

_NROW = 10000
_ZSTAGE = 624

_SHUF_DNUMS = lax.GatherDimensionNumbers(
    offset_dims=(), collapsed_slice_dims=(0,), start_index_map=(0,))


def _shuffle(v, perm):
    return lax.gather(v, perm[:, None], _SHUF_DNUMS, slice_sizes=(1,),
                      mode=lax.GatherScatterMode.PROMISE_IN_BOUNDS)


def _softmax_rows(n, src_rows, dst_rows, out_rows):
    @plsc.parallel_loop(0, n, unroll=2)
    def edge_body(e):
        ex = [jnp.exp(jnp.minimum(
            src_rows[e, pl.ds(j * _LANES, _LANES)]
            * dst_rows[e, pl.ds(j * _LANES, _LANES)], 80.0))
            for j in range(_VPR)]
        s = ex[0]
        for j in range(1, _VPR):
            s = s + ex[j]
        for sh in (8, 4, 2, 1):
            perm = lax.iota(jnp.int32, 16) ^ sh
            s = s + _shuffle(s, perm)
        r = 1.0 / s
        for j in range(_VPR):
            out_rows[e, pl.ds(j * _LANES, _LANES)] = ex[j] * r


def _sc_body(z_hbm, pidx_hbm, out_hbm,
             z_sp, pidx_v, si0, si1, di0, di1,
             sr0, sr1, dr0, dr1, or0, or1,
             ss0, ss1, sd0, sd1, sw0, sw1):
    sid = lax.axis_index("s")
    wid = sid * _NC + lax.axis_index("c")
    w_base = wid * _PER_W

    pltpu.sync_copy(z_hbm.at[pl.ds(sid * _ZSTAGE, _ZSTAGE)],
                    z_sp.at[pl.ds(sid * _ZSTAGE, _ZSTAGE)])

    @pl.when(sid == 0)
    def _():
        pltpu.sync_copy(z_hbm.at[pl.ds(_NS * _ZSTAGE, _NROW - _NS * _ZSTAGE)],
                        z_sp.at[pl.ds(_NS * _ZSTAGE, _NROW - _NS * _ZSTAGE)])

    pltpu.sync_copy(pidx_hbm.at[pl.ds(w_base, _PER_W)], pidx_v)
    plsc.subcore_barrier()

    src_idx = (si0, si1)
    dst_idx = (di0, di1)
    src_rows = (sr0, sr1)
    dst_rows = (dr0, dr1)
    out_rows = (or0, or1)
    sem_s = (ss0, ss1)
    sem_d = (sd0, sd1)
    sem_w = (sw0, sw1)

    def unpack_idx(i, b, n_vregs):
        for k in range(n_vregs):
            w = pidx_v[pl.ds(i * _CHUNK + k * _LANES, _LANES)]
            src_idx[b][pl.ds(k * _LANES, _LANES)] = w & 0xFFFF
            dst_idx[b][pl.ds(k * _LANES, _LANES)] = (
                lax.shift_right_logical(w, 16))

    def gather(i, b):
        unpack_idx(i, b, _IVPC)
        pltpu.make_async_copy(
            z_sp.at[src_idx[b]], src_rows[b], sem_s[b]).start()
        pltpu.make_async_copy(
            z_sp.at[dst_idx[b]], dst_rows[b], sem_d[b]).start()

    def wait_gather(b):
        pltpu.make_async_copy(
            z_sp.at[src_idx[b]], src_rows[b], sem_s[b]).wait()
        pltpu.make_async_copy(
            z_sp.at[dst_idx[b]], dst_rows[b], sem_d[b]).wait()

    def writeback(i, b):
        pltpu.make_async_copy(
            out_rows[b], out_hbm.at[pl.ds(w_base + i * _CHUNK, _CHUNK)],
            sem_w[b]).start()

    def wait_writeback(i, b):
        pltpu.make_async_copy(
            out_rows[b], out_hbm.at[pl.ds(w_base + i * _CHUNK, _CHUNK)],
            sem_w[b]).wait()

    gather(0, 0)

    def pair_body(p, carry):
        for b in (0, 1):
            i = 2 * p + b

            @pl.when(i + 1 < _NFULL)
            def _():
                gather(i + 1, 1 - b)

            wait_gather(b)

            @pl.when(i >= 2)
            def _():
                wait_writeback(i - 2, b)

            _softmax_rows(_CHUNK, src_rows[b], dst_rows[b], out_rows[b])
            writeback(i, b)
        return carry

    lax.fori_loop(0, _NFULL // 2, pair_body, 0)

    t_base = w_base + _NFULL * _CHUNK
    unpack_idx(_NFULL, 0, _TAIL // _LANES)
    pltpu.async_copy(
        z_sp.at[si0.at[pl.ds(0, _TAIL)]], sr0.at[pl.ds(0, _TAIL)], ss0).wait()
    pltpu.async_copy(
        z_sp.at[di0.at[pl.ds(0, _TAIL)]], dr0.at[pl.ds(0, _TAIL)], sd0).wait()
    wait_writeback(_NFULL - 2, 0)
    _softmax_rows(_TAIL, sr0, dr0, or0)
    pltpu.sync_copy(or0.at[pl.ds(0, _TAIL)], out_hbm.at[pl.ds(t_base, _TAIL)])
    wait_writeback(_NFULL - 1, 1)


def _decode(z, packed_idx):
    mesh = plsc.VectorSubcoreMesh(core_axis_name="c", subcore_axis_name="s",
                                  num_cores=_NC, num_subcores=_NS)
    return pl.kernel(
        _sc_body,
        out_type=jax.ShapeDtypeStruct((_E, _D), jnp.float32),
        mesh=mesh,
        scratch_types=[
            pltpu.VMEM_SHARED((_NROW, _D), jnp.float32),
            pltpu.VMEM((_PER_W,), jnp.int32),
            pltpu.VMEM((_CHUNK,), jnp.int32),
            pltpu.VMEM((_CHUNK,), jnp.int32),
            pltpu.VMEM((_CHUNK,), jnp.int32),
            pltpu.VMEM((_CHUNK,), jnp.int32),
            pltpu.VMEM((_CHUNK, _D), jnp.float32),
            pltpu.VMEM((_CHUNK, _D), jnp.float32),
            pltpu.VMEM((_CHUNK, _D), jnp.float32),
            pltpu.VMEM((_CHUNK, _D), jnp.float32),
            pltpu.VMEM((_CHUNK, _D), jnp.float32),
            pltpu.VMEM((_CHUNK, _D), jnp.float32),
            pltpu.SemaphoreType.DMA,
            pltpu.SemaphoreType.DMA,
            pltpu.SemaphoreType.DMA,
            pltpu.SemaphoreType.DMA,
            pltpu.SemaphoreType.DMA,
            pltpu.SemaphoreType.DMA,
        ],
    )(z, packed_idx)


def kernel(z, edge_index):
    ei = edge_index.astype(jnp.int32)
    packed = ei[0] | (ei[1] << 16)
    return _decode(z, packed)
